# baseline (speedup 1.0000x reference)
import functools
import math

import jax
import jax.numpy as jnp
from jax.experimental import pallas as pl

NH = 4
SDIM = 10
_INTERPRET = False


def _pc(kern, grid, in_specs, out_specs, out_shape):
    return pl.pallas_call(
        kern,
        grid=grid,
        in_specs=in_specs,
        out_specs=out_specs,
        out_shape=out_shape,
        interpret=_INTERPRET,
    )


def _dense(x, W, gamma=None, bias=None, res=None, pre=None, bm=256, bn=256,
           bk=None):
    M, K = x.shape
    N = W.shape[1]
    bm = min(bm, M)
    bn = min(bn, N)
    bk = K if bk is None else min(bk, K)
    if pre == 'ln':
        bk = K
    nk = K // bk
    grid = (M // bm, N // bn, nk)

    in_specs = [
        pl.BlockSpec((bm, bk), lambda i, j, k: (i, k)),
        pl.BlockSpec((bk, bn), lambda i, j, k: (k, j)),
    ]
    args = [x, W]
    if gamma is not None:
        in_specs.append(pl.BlockSpec((1, K), lambda i, j, k: (0, 0)))
        args.append(gamma.reshape(1, K))
    if bias is not None:
        in_specs.append(pl.BlockSpec((1, bn), lambda i, j, k: (0, j)))
        args.append(bias.reshape(1, N))
    if res is not None:
        in_specs.append(pl.BlockSpec((bm, bn), lambda i, j, k: (i, j)))
        args.append(res)

    def kern(*refs):
        x_ref, w_ref = refs[0], refs[1]
        out_ref = refs[-1]
        pos = 2
        g_ref = b_ref = r_ref = None
        if gamma is not None:
            g_ref = refs[pos]; pos += 1
        if bias is not None:
            b_ref = refs[pos]; pos += 1
        if res is not None:
            r_ref = refs[pos]; pos += 1

        xv = x_ref[...]
        if pre == 'ln':
            m = jnp.mean(xv, axis=-1, keepdims=True)
            v = jnp.mean((xv - m) ** 2, axis=-1, keepdims=True)
            xv = g_ref[...] * (xv - m) / jnp.sqrt(v + 1e-5)
        elif pre == 'gelu':
            xv = jax.nn.gelu(xv)
        acc = jnp.dot(xv, w_ref[...], preferred_element_type=jnp.float32)

        def tail(a):
            if b_ref is not None:
                a = a + b_ref[...]
            if r_ref is not None:
                a = a + r_ref[...]
            return a

        if nk == 1:
            out_ref[...] = tail(acc)
        else:
            k = pl.program_id(2)

            @pl.when(k == 0)
            def _():
                out_ref[...] = acc

            @pl.when(k > 0)
            def _():
                out_ref[...] += acc

            if bias is not None or res is not None:
                @pl.when(k == nk - 1)
                def _():
                    out_ref[...] = tail(out_ref[...])

    return _pc(kern, grid, in_specs,
               pl.BlockSpec((bm, bn), lambda i, j, k: (i, j)),
               jax.ShapeDtypeStruct((M, N), jnp.float32))(*args)


def _attention_qkv(qkv, nh, causal, scale, bq=256):
    B, T, C3 = qkv.shape
    C = C3 // 3
    hd = C // nh
    bq = min(bq, T)
    grid = (B, nh, T // bq)

    in_specs = [
        pl.BlockSpec((1, bq, hd), lambda b, h, i: (b, i, h)),
        pl.BlockSpec((1, T, hd), lambda b, h, i: (b, 0, nh + h)),
        pl.BlockSpec((1, T, hd), lambda b, h, i: (b, 0, 2 * nh + h)),
    ]

    def kern(q_ref, k_ref, v_ref, o_ref):
        i = pl.program_id(2)
        q = q_ref[0]
        k = k_ref[0]
        v = v_ref[0]
        s = jax.lax.dot_general(q, k, (((1,), (1,)), ((), ())),
                                preferred_element_type=jnp.float32)
        if scale != 1.0:
            s = s * scale
        if causal:
            row = i * bq + jax.lax.broadcasted_iota(jnp.int32, (bq, T), 0)
            col = jax.lax.broadcasted_iota(jnp.int32, (bq, T), 1)
            s = jnp.where(col <= row, s, jnp.float32(-1e9))
        m = jnp.max(s, axis=-1, keepdims=True)
        e = jnp.exp(s - m)
        p = e / jnp.sum(e, axis=-1, keepdims=True)
        o_ref[0, 0] = jnp.dot(p, v, preferred_element_type=jnp.float32)

    return _pc(kern, grid, in_specs,
               pl.BlockSpec((1, 1, bq, hd), lambda b, h, i: (b, h, i, 0)),
               jax.ShapeDtypeStruct((B, nh, T, hd), jnp.float32))(qkv, qkv, qkv)


def _attention_cross(q, k, v, nh, bq=256):
    B, Tq, C = q.shape
    Tk = k.shape[1]
    hd = C // nh
    bq = min(bq, Tq)
    grid = (B, nh, Tq // bq)

    in_specs = [
        pl.BlockSpec((1, bq, hd), lambda b, h, i: (b, i, h)),
        pl.BlockSpec((1, Tk, hd), lambda b, h, i: (b, 0, h)),
        pl.BlockSpec((1, Tk, hd), lambda b, h, i: (b, 0, h)),
    ]

    def kern(q_ref, k_ref, v_ref, o_ref):
        s = jax.lax.dot_general(q_ref[0], k_ref[0], (((1,), (1,)), ((), ())),
                                preferred_element_type=jnp.float32)
        m = jnp.max(s, axis=-1, keepdims=True)
        e = jnp.exp(s - m)
        p = e / jnp.sum(e, axis=-1, keepdims=True)
        o_ref[0, 0] = jnp.dot(p, v_ref[0], preferred_element_type=jnp.float32)

    return _pc(kern, grid, in_specs,
               pl.BlockSpec((1, 1, bq, hd), lambda b, h, i: (b, h, i, 0)),
               jax.ShapeDtypeStruct((B, nh, Tq, hd), jnp.float32))(q, k, v)


def _gate(x_next, scan):
    B, S, C = scan.shape

    def kern(xn_ref, sc_ref, o_ref):
        d = xn_ref[0] - sc_ref[0]
        g = jnp.sum(d * d, axis=-1, keepdims=True)
        t = jax.lax.broadcasted_iota(jnp.int32, (S, 1), 0)
        o_ref[0] = jnp.where(t < S - 1, g, jnp.float32(0.0))

    return _pc(kern, (B,),
               [pl.BlockSpec((1, S, C), lambda b: (b, 0, 0)),
                pl.BlockSpec((1, S, C), lambda b: (b, 0, 0))],
               pl.BlockSpec((1, S, 1), lambda b: (b, 0, 0)),
               jax.ShapeDtypeStruct((B, S, 1), jnp.float32))(x_next, scan)


def _rank_onehot(g_row, g_col, K):
    B, _, S = g_row.shape

    def kern(gr_ref, gc_ref, p_ref):
        grow = gr_ref[0]
        gcol = gc_ref[0]
        gt = (grow > gcol).astype(jnp.float32)
        ii = jax.lax.broadcasted_iota(jnp.int32, (S, S), 0)
        jj = jax.lax.broadcasted_iota(jnp.int32, (S, S), 1)
        tie = ((grow == gcol) & (jj < ii)).astype(jnp.float32)
        ones = jnp.ones((S, 1), jnp.float32)
        rank = jnp.dot(gt + tie, ones, preferred_element_type=jnp.float32)
        rr = jax.lax.broadcasted_iota(jnp.int32, (S, K), 1).astype(jnp.float32)
        p_ref[0] = (rank == rr).astype(jnp.float32)

    return _pc(kern, (B,),
               [pl.BlockSpec((1, 1, S), lambda b: (b, 0, 0)),
                pl.BlockSpec((1, S, 1), lambda b: (b, 0, 0))],
               pl.BlockSpec((1, S, K), lambda b: (b, 0, 0)),
               jax.ShapeDtypeStruct((B, S, K), jnp.float32))(g_row, g_col)


def _gather_mm(PT, g_col, values):
    B, K, S = PT.shape
    C = values.shape[2]

    def kern(pt_ref, gc_ref, v_ref, o_ref):
        gc = gc_ref[0]
        m = jnp.max(gc, axis=0, keepdims=True)
        e = jnp.exp(gc - m)
        sm_g = e / jnp.sum(e, axis=0, keepdims=True)
        pt = pt_ref[0]
        w = jnp.dot(pt, sm_g, preferred_element_type=jnp.float32)
        vals = v_ref[0]
        vm = jnp.max(vals, axis=-1, keepdims=True)
        ve = jnp.exp(vals - vm)
        sv = ve / jnp.sum(ve, axis=-1, keepdims=True)
        g = jnp.dot(pt, sv, preferred_element_type=jnp.float32)
        o_ref[0] = g * w

    return _pc(kern, (B,),
               [pl.BlockSpec((1, K, S), lambda b: (b, 0, 0)),
                pl.BlockSpec((1, S, 1), lambda b: (b, 0, 0)),
                pl.BlockSpec((1, S, C), lambda b: (b, 0, 0))],
               pl.BlockSpec((1, K, C), lambda b: (b, 0, 0)),
               jax.ShapeDtypeStruct((B, K, C), jnp.float32))(PT, g_col, values)


def _scatter_mm(P, PT, g_col, values, Wpos, bpos, bs=512):
    B, S, K = P.shape
    C = values.shape[2]
    bs = min(bs, S)
    grid = (B, S // bs)

    def kern(p_ref, pt_ref, gc_ref, v_ref, wp_ref, bp_ref, o_ref):
        gc = gc_ref[0]
        m = jnp.max(gc, axis=0, keepdims=True)
        e = jnp.exp(gc - m)
        sm_g = e / jnp.sum(e, axis=0, keepdims=True)
        w = jnp.dot(pt_ref[0], sm_g, preferred_element_type=jnp.float32)
        vw = v_ref[0] * w
        o_ref[0] = (jnp.dot(p_ref[0], vw, preferred_element_type=jnp.float32)
                    + wp_ref[...] + bp_ref[...])

    return _pc(kern, grid,
               [pl.BlockSpec((1, bs, K), lambda b, i: (b, i, 0)),
                pl.BlockSpec((1, K, S), lambda b, i: (b, 0, 0)),
                pl.BlockSpec((1, S, 1), lambda b, i: (b, 0, 0)),
                pl.BlockSpec((1, K, C), lambda b, i: (b, 0, 0)),
                pl.BlockSpec((bs, C), lambda b, i: (i, 0)),
                pl.BlockSpec((1, C), lambda b, i: (0, 0))],
               pl.BlockSpec((1, bs, C), lambda b, i: (b, i, 0)),
               jax.ShapeDtypeStruct((B, S, C), jnp.float32))(
                   P, PT, g_col, values, Wpos, bpos.reshape(1, C))


def _block(x, ln1, ln2, Wqkv, Wo, Wfc, Wproj):
    B, T, C = x.shape
    hd = C // NH
    xf = x.reshape(B * T, C)
    qkv = _dense(xf, Wqkv, gamma=ln1, pre='ln').reshape(B, T, 3 * C)
    y = _attention_qkv(qkv, NH, causal=True, scale=1.0 / math.sqrt(hd))
    yf = y.transpose(0, 2, 1, 3).reshape(B * T, C)
    h = _dense(yf, Wo, res=xf)
    h2 = _dense(h, Wfc, gamma=ln2, pre='ln')
    out = _dense(h2, Wproj, pre='gelu', res=h, bk=2048)
    return out.reshape(B, T, C)


def kernel(x, up_ln1, up_ln2, up_Wqkv, up_Wo, up_Wfc, up_Wproj,
           down_ln1, down_ln2, down_Wqkv, down_Wo, down_Wfc, down_Wproj,
           Wup, bup, Wpos, bpos, Wsg, bsg,
           ds_ln1, ds_ln2, ds_ln3, ds_Wq, ds_Wk, ds_Wv, ds_Wo,
           ds_Wfc, ds_Wproj, Wdown, bdown):
    B, T, C = x.shape
    C2 = Wup.shape[1]
    Kin = Wsg.shape[0] // SDIM
    S = Wsg.shape[1]

    scan = _block(x, up_ln1, up_ln2, up_Wqkv, up_Wo, up_Wfc, up_Wproj)

    x_next = jnp.concatenate(
        [x[:, 1:, :], jnp.zeros((B, 1, C), x.dtype)], axis=1)
    g_col = _gate(x_next, scan)
    g_row = g_col.reshape(B, 1, T)
    P = _rank_onehot(g_row, g_col, Kin)
    PT = P.transpose(0, 2, 1)
    gathered = _gather_mm(PT, g_col, scan)

    p_up = _dense(gathered.reshape(B * Kin, C), Wup, bias=bup)
    passed = p_up.reshape(B, Kin, C2)

    sg_in = passed[:, :, :SDIM].reshape(B, Kin * SDIM)
    sg_in8 = jnp.concatenate(
        [sg_in, jnp.zeros((8 - B, Kin * SDIM), sg_in.dtype)], axis=0)
    sg = _dense(sg_in8, Wsg, bias=bsg, bm=8)[:B]
    sg_col = sg.reshape(B, S, 1)
    sg_row = sg.reshape(B, 1, S)
    P2 = _rank_onehot(sg_row, sg_col, Kin)
    PT2 = P2.transpose(0, 2, 1)
    scattered = _scatter_mm(P2, PT2, sg_col, passed, Wpos, bpos)

    sf = scattered.reshape(B * S, C2)
    pf = passed.reshape(B * Kin, C2)
    q = _dense(sf, ds_Wq, gamma=ds_ln2, pre='ln').reshape(B, S, C2)
    kk = _dense(pf, ds_Wk, gamma=ds_ln1, pre='ln').reshape(B, Kin, C2)
    vv = _dense(pf, ds_Wv, gamma=ds_ln1, pre='ln').reshape(B, Kin, C2)
    y = _attention_cross(q, kk, vv, NH)
    yf = y.transpose(0, 2, 1, 3).reshape(B * S, C2)
    sx = _dense(yf, ds_Wo, res=sf)
    h = _dense(sx, ds_Wfc, gamma=ds_ln3, pre='ln')
    pds = _dense(h, ds_Wproj, pre='gelu', res=sx, bk=2048)

    pd = _dense(pds, Wdown, bias=bdown, res=scan.reshape(B * T, C))
    out = _block(pd.reshape(B, T, C), down_ln1, down_ln2,
                 down_Wqkv, down_Wo, down_Wfc, down_Wproj)

    loss = jnp.zeros((1,), x.dtype)
    return out, loss


# speedup vs baseline: 2.3282x; 2.3282x over previous
import functools
import math

import jax
import jax.numpy as jnp
from jax.experimental import pallas as pl

NH = 4
SDIM = 10
_INTERPRET = False


def _pc(kern, grid, in_specs, out_specs, out_shape):
    return pl.pallas_call(
        kern,
        grid=grid,
        in_specs=in_specs,
        out_specs=out_specs,
        out_shape=out_shape,
        interpret=_INTERPRET,
    )


def _mxdot(a, b):
    return jnp.dot(a.astype(jnp.bfloat16), b.astype(jnp.bfloat16),
                   preferred_element_type=jnp.float32)


def _dense(x, W, gamma=None, bias=None, res=None, pre=None, bm=1024, bn=1024,
           bk=None, out_dtype=jnp.float32):
    M, K = x.shape
    N = W.shape[1]
    bm = min(bm, M)
    bn = min(bn, N)
    if bk is None:
        bk = K if K <= 2048 else 512
    bk = min(bk, K)
    if pre == 'ln':
        bk = K
    nk = K // bk
    grid = (M // bm, N // bn, nk)
    W = W.astype(jnp.bfloat16)

    in_specs = [
        pl.BlockSpec((bm, bk), lambda i, j, k: (i, k)),
        pl.BlockSpec((bk, bn), lambda i, j, k: (k, j)),
    ]
    args = [x, W]
    if gamma is not None:
        in_specs.append(pl.BlockSpec((1, K), lambda i, j, k: (0, 0)))
        args.append(gamma.reshape(1, K))
    if bias is not None:
        in_specs.append(pl.BlockSpec((1, bn), lambda i, j, k: (0, j)))
        args.append(bias.reshape(1, N))
    if res is not None:
        in_specs.append(pl.BlockSpec((bm, bn), lambda i, j, k: (i, j)))
        args.append(res)

    def kern(*refs):
        x_ref, w_ref = refs[0], refs[1]
        out_ref = refs[-1]
        pos = 2
        g_ref = b_ref = r_ref = None
        if gamma is not None:
            g_ref = refs[pos]; pos += 1
        if bias is not None:
            b_ref = refs[pos]; pos += 1
        if res is not None:
            r_ref = refs[pos]; pos += 1

        xv = x_ref[...]
        if pre == 'ln':
            xv = xv.astype(jnp.float32)
            m = jnp.mean(xv, axis=-1, keepdims=True)
            v = jnp.mean((xv - m) ** 2, axis=-1, keepdims=True)
            xv = g_ref[...] * (xv - m) / jnp.sqrt(v + 1e-5)
        elif pre == 'gelu':
            xv = jax.nn.gelu(xv.astype(jnp.float32))
        acc = _mxdot(xv, w_ref[...])

        def tail(a):
            if b_ref is not None:
                a = a + b_ref[...]
            if r_ref is not None:
                a = a + r_ref[...]
            return a

        if nk == 1:
            out_ref[...] = tail(acc).astype(out_dtype)
        else:
            k = pl.program_id(2)

            @pl.when(k == 0)
            def _():
                out_ref[...] = acc

            @pl.when(k > 0)
            def _():
                out_ref[...] += acc

            if bias is not None or res is not None:
                @pl.when(k == nk - 1)
                def _():
                    out_ref[...] = tail(out_ref[...])

    return _pc(kern, grid, in_specs,
               pl.BlockSpec((bm, bn), lambda i, j, k: (i, j)),
               jax.ShapeDtypeStruct((M, N), out_dtype))(*args)


def _attention_qkv(qkv, nh, causal, scale, bq=256):
    B, T, C3 = qkv.shape
    C = C3 // 3
    hd = C // nh
    bq = min(bq, T)
    grid = (B, nh, T // bq)

    in_specs = [
        pl.BlockSpec((1, bq, hd), lambda b, h, i: (b, i, h)),
        pl.BlockSpec((1, T, hd), lambda b, h, i: (b, 0, nh + h)),
        pl.BlockSpec((1, T, hd), lambda b, h, i: (b, 0, 2 * nh + h)),
    ]

    def kern(q_ref, k_ref, v_ref, o_ref):
        i = pl.program_id(2)
        q = q_ref[0].astype(jnp.bfloat16)
        k = k_ref[0].astype(jnp.bfloat16)
        v = v_ref[0]
        s = jax.lax.dot_general(q, k, (((1,), (1,)), ((), ())),
                                preferred_element_type=jnp.float32)
        if scale != 1.0:
            s = s * scale
        if causal:
            row = i * bq + jax.lax.broadcasted_iota(jnp.int32, (bq, T), 0)
            col = jax.lax.broadcasted_iota(jnp.int32, (bq, T), 1)
            s = jnp.where(col <= row, s, jnp.float32(-1e9))
        m = jnp.max(s, axis=-1, keepdims=True)
        e = jnp.exp(s - m)
        p = e / jnp.sum(e, axis=-1, keepdims=True)
        o_ref[0, 0] = _mxdot(p, v)

    return _pc(kern, grid, in_specs,
               pl.BlockSpec((1, 1, bq, hd), lambda b, h, i: (b, h, i, 0)),
               jax.ShapeDtypeStruct((B, nh, T, hd), jnp.float32))(qkv, qkv, qkv)


def _attention_cross(q, k, v, nh, bq=256):
    B, Tq, C = q.shape
    Tk = k.shape[1]
    hd = C // nh
    bq = min(bq, Tq)
    grid = (B, nh, Tq // bq)

    in_specs = [
        pl.BlockSpec((1, bq, hd), lambda b, h, i: (b, i, h)),
        pl.BlockSpec((1, Tk, hd), lambda b, h, i: (b, 0, h)),
        pl.BlockSpec((1, Tk, hd), lambda b, h, i: (b, 0, h)),
    ]

    def kern(q_ref, k_ref, v_ref, o_ref):
        s = jax.lax.dot_general(q_ref[0].astype(jnp.bfloat16),
                                k_ref[0].astype(jnp.bfloat16),
                                (((1,), (1,)), ((), ())),
                                preferred_element_type=jnp.float32)
        m = jnp.max(s, axis=-1, keepdims=True)
        e = jnp.exp(s - m)
        p = e / jnp.sum(e, axis=-1, keepdims=True)
        o_ref[0, 0] = _mxdot(p, v_ref[0])

    return _pc(kern, grid, in_specs,
               pl.BlockSpec((1, 1, bq, hd), lambda b, h, i: (b, h, i, 0)),
               jax.ShapeDtypeStruct((B, nh, Tq, hd), jnp.float32))(q, k, v)


def _gate(x_next, scan):
    B, S, C = scan.shape

    def kern(xn_ref, sc_ref, o_ref):
        d = xn_ref[0] - sc_ref[0]
        g = jnp.sum(d * d, axis=-1, keepdims=True)
        t = jax.lax.broadcasted_iota(jnp.int32, (S, 1), 0)
        o_ref[0] = jnp.where(t < S - 1, g, jnp.float32(0.0))

    return _pc(kern, (B,),
               [pl.BlockSpec((1, S, C), lambda b: (b, 0, 0)),
                pl.BlockSpec((1, S, C), lambda b: (b, 0, 0))],
               pl.BlockSpec((1, S, 1), lambda b: (b, 0, 0)),
               jax.ShapeDtypeStruct((B, S, 1), jnp.float32))(x_next, scan)


def _rank_onehot(g_row, g_col, K):
    B, _, S = g_row.shape

    def kern(gr_ref, gc_ref, p_ref):
        grow = gr_ref[0]
        gcol = gc_ref[0]
        gt = (grow > gcol).astype(jnp.float32)
        ii = jax.lax.broadcasted_iota(jnp.int32, (S, S), 0)
        jj = jax.lax.broadcasted_iota(jnp.int32, (S, S), 1)
        tie = ((grow == gcol) & (jj < ii)).astype(jnp.float32)
        ones = jnp.ones((S, 1), jnp.float32)
        rank = jnp.dot(gt + tie, ones, preferred_element_type=jnp.float32)
        rr = jax.lax.broadcasted_iota(jnp.int32, (S, K), 1).astype(jnp.float32)
        p_ref[0] = (rank == rr).astype(jnp.float32)

    return _pc(kern, (B,),
               [pl.BlockSpec((1, 1, S), lambda b: (b, 0, 0)),
                pl.BlockSpec((1, S, 1), lambda b: (b, 0, 0))],
               pl.BlockSpec((1, S, K), lambda b: (b, 0, 0)),
               jax.ShapeDtypeStruct((B, S, K), jnp.float32))(g_row, g_col)


def _gather_mm(PT, g_col, values):
    B, K, S = PT.shape
    C = values.shape[2]

    def kern(pt_ref, gc_ref, v_ref, o_ref):
        gc = gc_ref[0]
        m = jnp.max(gc, axis=0, keepdims=True)
        e = jnp.exp(gc - m)
        sm_g = e / jnp.sum(e, axis=0, keepdims=True)
        pt = pt_ref[0]
        w = jnp.dot(pt, sm_g, preferred_element_type=jnp.float32)
        vals = v_ref[0]
        vm = jnp.max(vals, axis=-1, keepdims=True)
        ve = jnp.exp(vals - vm)
        sv = ve / jnp.sum(ve, axis=-1, keepdims=True)
        g = _mxdot(pt, sv)
        o_ref[0] = g * w

    return _pc(kern, (B,),
               [pl.BlockSpec((1, K, S), lambda b: (b, 0, 0)),
                pl.BlockSpec((1, S, 1), lambda b: (b, 0, 0)),
                pl.BlockSpec((1, S, C), lambda b: (b, 0, 0))],
               pl.BlockSpec((1, K, C), lambda b: (b, 0, 0)),
               jax.ShapeDtypeStruct((B, K, C), jnp.float32))(PT, g_col, values)


def _scatter_mm(P, PT, g_col, values, Wpos, bpos, bs=512):
    B, S, K = P.shape
    C = values.shape[2]
    bs = min(bs, S)
    grid = (B, S // bs)

    def kern(p_ref, pt_ref, gc_ref, v_ref, wp_ref, bp_ref, o_ref):
        gc = gc_ref[0]
        m = jnp.max(gc, axis=0, keepdims=True)
        e = jnp.exp(gc - m)
        sm_g = e / jnp.sum(e, axis=0, keepdims=True)
        w = jnp.dot(pt_ref[0], sm_g, preferred_element_type=jnp.float32)
        vw = v_ref[0] * w
        o_ref[0] = _mxdot(p_ref[0], vw) + wp_ref[...] + bp_ref[...]

    return _pc(kern, grid,
               [pl.BlockSpec((1, bs, K), lambda b, i: (b, i, 0)),
                pl.BlockSpec((1, K, S), lambda b, i: (b, 0, 0)),
                pl.BlockSpec((1, S, 1), lambda b, i: (b, 0, 0)),
                pl.BlockSpec((1, K, C), lambda b, i: (b, 0, 0)),
                pl.BlockSpec((bs, C), lambda b, i: (i, 0)),
                pl.BlockSpec((1, C), lambda b, i: (0, 0))],
               pl.BlockSpec((1, bs, C), lambda b, i: (b, i, 0)),
               jax.ShapeDtypeStruct((B, S, C), jnp.float32))(
                   P, PT, g_col, values, Wpos, bpos.reshape(1, C))


def _block(x, ln1, ln2, Wqkv, Wo, Wfc, Wproj):
    B, T, C = x.shape
    hd = C // NH
    xf = x.reshape(B * T, C)
    qkv = _dense(xf, Wqkv, gamma=ln1, pre='ln').reshape(B, T, 3 * C)
    y = _attention_qkv(qkv, NH, causal=True, scale=1.0 / math.sqrt(hd))
    yf = y.transpose(0, 2, 1, 3).reshape(B * T, C)
    h = _dense(yf, Wo, res=xf)
    h2 = _dense(h, Wfc, gamma=ln2, pre='ln', out_dtype=jnp.bfloat16)
    out = _dense(h2, Wproj, pre='gelu', res=h)
    return out.reshape(B, T, C)


def kernel(x, up_ln1, up_ln2, up_Wqkv, up_Wo, up_Wfc, up_Wproj,
           down_ln1, down_ln2, down_Wqkv, down_Wo, down_Wfc, down_Wproj,
           Wup, bup, Wpos, bpos, Wsg, bsg,
           ds_ln1, ds_ln2, ds_ln3, ds_Wq, ds_Wk, ds_Wv, ds_Wo,
           ds_Wfc, ds_Wproj, Wdown, bdown):
    B, T, C = x.shape
    C2 = Wup.shape[1]
    Kin = Wsg.shape[0] // SDIM
    S = Wsg.shape[1]

    scan = _block(x, up_ln1, up_ln2, up_Wqkv, up_Wo, up_Wfc, up_Wproj)

    x_next = jnp.concatenate(
        [x[:, 1:, :], jnp.zeros((B, 1, C), x.dtype)], axis=1)
    g_col = _gate(x_next, scan)
    g_row = g_col.reshape(B, 1, T)
    P = _rank_onehot(g_row, g_col, Kin)
    PT = P.transpose(0, 2, 1)
    gathered = _gather_mm(PT, g_col, scan)

    p_up = _dense(gathered.reshape(B * Kin, C), Wup, bias=bup)
    passed = p_up.reshape(B, Kin, C2)

    sg_in = passed[:, :, :SDIM].reshape(B, Kin * SDIM)
    sg_in8 = jnp.concatenate(
        [sg_in, jnp.zeros((8 - B, Kin * SDIM), sg_in.dtype)], axis=0)
    sg = _dense(sg_in8, Wsg, bias=bsg, bm=8)[:B]
    sg_col = sg.reshape(B, S, 1)
    sg_row = sg.reshape(B, 1, S)
    P2 = _rank_onehot(sg_row, sg_col, Kin)
    PT2 = P2.transpose(0, 2, 1)
    scattered = _scatter_mm(P2, PT2, sg_col, passed, Wpos, bpos)

    sf = scattered.reshape(B * S, C2)
    pf = passed.reshape(B * Kin, C2)
    q = _dense(sf, ds_Wq, gamma=ds_ln2, pre='ln').reshape(B, S, C2)
    kk = _dense(pf, ds_Wk, gamma=ds_ln1, pre='ln').reshape(B, Kin, C2)
    vv = _dense(pf, ds_Wv, gamma=ds_ln1, pre='ln').reshape(B, Kin, C2)
    y = _attention_cross(q, kk, vv, NH)
    yf = y.transpose(0, 2, 1, 3).reshape(B * S, C2)
    sx = _dense(yf, ds_Wo, res=sf)
    h = _dense(sx, ds_Wfc, gamma=ds_ln3, pre='ln', bn=2048,
               out_dtype=jnp.bfloat16)
    pds = _dense(h, ds_Wproj, pre='gelu', res=sx, bn=2048)

    pd = _dense(pds, Wdown, bias=bdown, res=scan.reshape(B * T, C))
    out = _block(pd.reshape(B, T, C), down_ln1, down_ln2,
                 down_Wqkv, down_Wo, down_Wfc, down_Wproj)

    loss = jnp.zeros((1,), x.dtype)
    return out, loss
